# baseline (speedup 1.0000x reference)
import numpy as np
import jax
import jax.numpy as jnp
from jax.experimental import pallas as pl
from jax.experimental.pallas import tpu as pltpu

_B = 128


def _sel1():
    R = np.zeros((5, 2, 4, 2, 2), np.float32)
    for u in range(2):
        for r in range(4):
            for y in range(2):
                for h in range(2):
                    d = 4 * u + r - 2 * y - h
                    if 0 <= d < 5:
                        R[d, u, r, y, h] = 1.0
    C = np.zeros((5, 32, 2, 16), np.float32)
    for w in range(32):
        for p in range(2):
            for v in range(14):
                d = w - 2 * v - p
                if 0 <= d < 5:
                    C[d, w, p, v] = 1.0
    return np.einsum("duryh,jwpv->djurwyhpv", R, C)


def _sel2():
    R = np.zeros((5, 3, 2, 2), np.float32)
    for t in range(3):
        for y in range(2):
            for h in range(2):
                d = 2 * t + y - h
                if 0 <= d < 5:
                    R[d, t, y, h] = 1.0
    C = np.zeros((5, 16, 2, 8), np.float32)
    for w in range(14):
        for p in range(2):
            for v in range(5):
                d = w - 2 * v - p
                if 0 <= d < 5:
                    C[d, w, p, v] = 1.0
    return np.einsum("dtyh,jwpv->djtywhpv", R, C)


_TS1 = _sel1()
_TS2 = _sel2()
_BM1 = np.repeat((np.arange(32) % 16 < 14).astype(np.float32), 8)[None, :]
_BM2 = np.repeat((np.arange(8) < 5).astype(np.float32), 16)[None, :]


def _lenet_kernel(x_ref, wq1_ref, b1_ref, wq2_ref, b2_ref,
                  w3_ref, b3_ref, w4_ref, b4_ref, w5_ref, b5_ref, o_ref):
    B = x_ref.shape[0]

    piece = [[None] * 8 for _ in range(3)]
    for ci in range(3):
        for m in range(4):
            q = x_ref[:, ci, 8 * m:8 * m + 8, :].astype(jnp.bfloat16)
            q = q.reshape(B, 256)
            piece[ci][2 * m] = q[:, :128]
            piece[ci][2 * m + 1] = q[:, 128:]

    a1js = []
    for j in range(7):
        lhs = jnp.concatenate(
            [piece[ci][j + u] for u in range(2) for ci in range(3)], axis=1)
        y = jnp.dot(lhs, wq1_ref[...], preferred_element_type=jnp.float32)
        h0 = jnp.maximum(jnp.maximum(y[:, 0:128], y[:, 128:256]),
                         jnp.maximum(y[:, 256:384], y[:, 384:512]))
        h1 = jnp.maximum(jnp.maximum(y[:, 512:640], y[:, 640:768]),
                         jnp.maximum(y[:, 768:896], y[:, 896:1024]))
        a1 = jnp.maximum(jnp.concatenate([h0, h1], axis=1) + b1_ref[...], 0.0)
        a1js.append(a1.astype(jnp.bfloat16))
    a1s = jnp.stack(a1js, axis=0)

    y2 = None
    for t in range(3):
        p = jnp.dot(a1s[t:t + 5].reshape(5 * B, 256), wq2_ref[t],
                    preferred_element_type=jnp.float32)
        y2 = p if y2 is None else y2 + p
    a2 = jnp.maximum(jnp.maximum(y2[:, 0:128], y2[:, 128:256]),
                     jnp.maximum(y2[:, 256:384], y2[:, 384:512]))
    a2 = jnp.maximum(a2 + b2_ref[...], 0.0).reshape(5, B, 128)

    f_in = jnp.concatenate([a2[h] for h in range(5)],
                           axis=1).astype(jnp.bfloat16)
    f1 = jnp.maximum(jnp.dot(f_in, w3_ref[...],
                             preferred_element_type=jnp.float32)
                     + b3_ref[...], 0.0)
    f2 = jnp.maximum(jnp.dot(f1.astype(jnp.bfloat16), w4_ref[...],
                             preferred_element_type=jnp.float32)
                     + b4_ref[...], 0.0)
    logits = jnp.dot(f2.astype(jnp.bfloat16), w5_ref[...],
                     preferred_element_type=jnp.float32) + b5_ref[...]
    o_ref[...] = logits[:, :100]


def kernel(x, w1, b1, w2, b2, w3, b3, w4, b4, w5, b5):
    n = x.shape[0]

    w1t = w1[:, :8].reshape(5, 5, 3, 8)
    t1 = jnp.tensordot(w1t, _TS1, axes=[[0, 1], [0, 1]])
    wq1 = t1.transpose(2, 0, 3, 4, 5, 6, 7, 8, 1).astype(
        jnp.bfloat16).reshape(768, 1024)
    w2t = jnp.pad(w2[:, :16].reshape(5, 5, 6, 16),
                  ((0, 0), (0, 0), (0, 2), (0, 0)))
    t2 = jnp.tensordot(w2t, _TS2, axes=[[0, 1], [0, 1]])
    wq2 = t2.transpose(2, 3, 4, 0, 5, 6, 7, 1).astype(
        jnp.bfloat16).reshape(3, 256, 512)
    w3c = jnp.pad(w3.reshape(5, 80, 128),
                  ((0, 0), (0, 48), (0, 0))).reshape(640, 128).astype(
                      jnp.bfloat16)
    b1L = jnp.tile(b1[:, :8], (1, 32)) * _BM1
    b2L = jnp.tile(b2[:, :16], (1, 8)) * _BM2

    c2 = lambda i: (0, 0)
    c3 = lambda i: (0, 0, 0)
    out = pl.pallas_call(
        _lenet_kernel,
        out_shape=jax.ShapeDtypeStruct((n, 100), jnp.float32),
        grid=(n // _B,),
        in_specs=[
            pl.BlockSpec((_B, 3, 32, 32), lambda i: (i, 0, 0, 0)),
            pl.BlockSpec((768, 1024), c2),
            pl.BlockSpec((1, 256), c2),
            pl.BlockSpec((3, 256, 512), c3),
            pl.BlockSpec((1, 128), c2),
            pl.BlockSpec((640, 128), c2),
            pl.BlockSpec((1, 128), c2),
            pl.BlockSpec((128, 128), c2),
            pl.BlockSpec((1, 128), c2),
            pl.BlockSpec((128, 128), c2),
            pl.BlockSpec((1, 128), c2),
        ],
        out_specs=pl.BlockSpec((_B, 100), lambda i: (i, 0)),
        compiler_params=pltpu.CompilerParams(
            dimension_semantics=("parallel",),
            vmem_limit_bytes=100 * 1024 * 1024,
        ),
    )(x, wq1, b1L, wq2, b2L, w3c, b3,
      w4.astype(jnp.bfloat16), b4, w5.astype(jnp.bfloat16), b5)

    return out


# speedup vs baseline: 1.3726x; 1.3726x over previous
import numpy as np
import jax
import jax.numpy as jnp
from jax.experimental import pallas as pl
from jax.experimental.pallas import tpu as pltpu

_B = 128


def _sel1():
    R = np.zeros((5, 2, 4, 2, 2), np.float32)
    for u in range(2):
        for r in range(4):
            for y in range(2):
                for h in range(2):
                    d = 4 * u + r - 2 * y - h
                    if 0 <= d < 5:
                        R[d, u, r, y, h] = 1.0
    C = np.zeros((5, 32, 2, 16), np.float32)
    for w in range(32):
        for p in range(2):
            for v in range(14):
                d = w - 2 * v - p
                if 0 <= d < 5:
                    C[d, w, p, v] = 1.0
    return np.einsum("duryh,jwpv->djurwyhpv", R, C)


def _sel2():
    R = np.zeros((5, 3, 2, 2), np.float32)
    for t in range(3):
        for y in range(2):
            for h in range(2):
                d = 2 * t + y - h
                if 0 <= d < 5:
                    R[d, t, y, h] = 1.0
    C = np.zeros((5, 16, 2, 8), np.float32)
    for w in range(14):
        for p in range(2):
            for v in range(5):
                d = w - 2 * v - p
                if 0 <= d < 5:
                    C[d, w, p, v] = 1.0
    return np.einsum("dtyh,jwpv->djtywhpv", R, C)


_TS1 = _sel1()
_TS2 = _sel2()
_BM1 = np.repeat((np.arange(32) % 16 < 14).astype(np.float32), 8)[None, :]
_BM2 = np.repeat((np.arange(8) < 5).astype(np.float32), 16)[None, :]
_Z48 = np.zeros((48, 128), np.float32)


def _lenet_kernel(x_ref, wq1_ref, b1_ref, wq2_ref, b2_ref,
                  w3_ref, b3_ref, w4_ref, b4_ref, w5_ref, b5_ref, o_ref):
    B = x_ref.shape[0]

    a1js = []
    for j in range(7):
        lhs = jnp.concatenate(
            [x_ref[:, ci, (j + u) * 128:(j + u + 1) * 128]
             for u in range(2) for ci in range(3)], axis=1)
        y = jnp.dot(lhs, wq1_ref[...], preferred_element_type=jnp.float32)
        h0 = jnp.maximum(jnp.maximum(y[:, 0:128], y[:, 128:256]),
                         jnp.maximum(y[:, 256:384], y[:, 384:512]))
        h1 = jnp.maximum(jnp.maximum(y[:, 512:640], y[:, 640:768]),
                         jnp.maximum(y[:, 768:896], y[:, 896:1024]))
        a1 = jnp.maximum(jnp.concatenate([h0, h1], axis=1) + b1_ref[...], 0.0)
        a1js.append(a1.astype(jnp.bfloat16))
    a1s = jnp.stack(a1js, axis=0)

    y2 = None
    for t in range(3):
        p = jnp.dot(a1s[t:t + 5].reshape(5 * B, 256), wq2_ref[t],
                    preferred_element_type=jnp.float32)
        y2 = p if y2 is None else y2 + p
    a2 = jnp.maximum(jnp.maximum(y2[:, 0:128], y2[:, 128:256]),
                     jnp.maximum(y2[:, 256:384], y2[:, 384:512]))
    a2 = jnp.maximum(a2 + b2_ref[...], 0.0).reshape(5, B, 128)

    facc = b3_ref[...]
    for h in range(5):
        facc = facc + jnp.dot(a2[h].astype(jnp.bfloat16), w3_ref[h],
                              preferred_element_type=jnp.float32)
    f1 = jnp.maximum(facc, 0.0)
    f2 = jnp.maximum(jnp.dot(f1.astype(jnp.bfloat16), w4_ref[...],
                             preferred_element_type=jnp.float32)
                     + b4_ref[...], 0.0)
    logits = jnp.dot(f2.astype(jnp.bfloat16), w5_ref[...],
                     preferred_element_type=jnp.float32) + b5_ref[...]
    o_ref[...] = logits[:, :100]


def kernel(x, w1, b1, w2, b2, w3, b3, w4, b4, w5, b5):
    n = x.shape[0]
    xs = x.astype(jnp.bfloat16).reshape(n, 3, 1024)

    w1t = w1[:, :8].reshape(5, 5, 3, 8)
    wq1 = jnp.einsum("djco,djurwyhpv->ucrwyhpvo", w1t, _TS1,
                     preferred_element_type=jnp.bfloat16).reshape(768, 1024)
    w2t = jnp.pad(w2[:, :16].reshape(5, 5, 6, 16),
                  ((0, 0), (0, 0), (0, 2), (0, 0)))
    wq2 = jnp.einsum("djco,djtywhpv->tywchpvo", w2t, _TS2,
                     preferred_element_type=jnp.bfloat16).reshape(3, 256, 512)
    w3c = jnp.pad(w3.reshape(5, 80, 128),
                  ((0, 0), (0, 48), (0, 0))).astype(jnp.bfloat16)
    b1L = jnp.tile(b1[:, :8], (1, 32)) * _BM1
    b2L = jnp.tile(b2[:, :16], (1, 8)) * _BM2

    c2 = lambda i: (0, 0)
    c3 = lambda i: (0, 0, 0)
    c4 = lambda i: (0, 0, 0, 0)
    out = pl.pallas_call(
        _lenet_kernel,
        out_shape=jax.ShapeDtypeStruct((n, 100), jnp.float32),
        grid=(n // _B,),
        in_specs=[
            pl.BlockSpec((_B, 3, 1024), lambda i: (i, 0, 0)),
            pl.BlockSpec((768, 1024), c2),
            pl.BlockSpec((1, 256), c2),
            pl.BlockSpec((3, 256, 512), c3),
            pl.BlockSpec((1, 128), c2),
            pl.BlockSpec((5, 128, 128), c3),
            pl.BlockSpec((1, 128), c2),
            pl.BlockSpec((128, 128), c2),
            pl.BlockSpec((1, 128), c2),
            pl.BlockSpec((128, 128), c2),
            pl.BlockSpec((1, 128), c2),
        ],
        out_specs=pl.BlockSpec((_B, 100), lambda i: (i, 0)),
        compiler_params=pltpu.CompilerParams(
            dimension_semantics=("parallel",),
            vmem_limit_bytes=100 * 1024 * 1024,
        ),
    )(xs, wq1, b1L, wq2, b2L, w3c, b3,
      w4.astype(jnp.bfloat16), b4, w5.astype(jnp.bfloat16), b5)

    return out


# speedup vs baseline: 1.6435x; 1.1974x over previous
import numpy as np
import jax
import jax.numpy as jnp
from jax.experimental import pallas as pl
from jax.experimental.pallas import tpu as pltpu

_B = 128


def _sel1():
    R = np.zeros((5, 2, 4, 2, 2), np.float32)
    for u in range(2):
        for r in range(4):
            for r2 in range(2):
                for hp in range(2):
                    di = 4 * u + r - 2 * r2 - hp
                    if 0 <= di < 5:
                        R[di, u, r, r2, hp] = 1.0
    C = np.zeros((5, 32, 2, 16), np.float32)
    for w_in in range(32):
        for wp in range(2):
            for w2 in range(14):
                dj = w_in - 2 * w2 - wp
                if 0 <= dj < 5:
                    C[dj, w_in, wp, w2] = 1.0
    return R, C


def _sel2():
    R = np.zeros((5, 3, 2, 2), np.float32)
    for t in range(3):
        for r2 in range(2):
            for hp in range(2):
                di = 2 * t + r2 - hp
                if 0 <= di < 5:
                    R[di, t, r2, hp] = 1.0
    C = np.zeros((5, 16, 2, 8), np.float32)
    for w in range(14):
        for wp in range(2):
            for w2 in range(5):
                dj = w - 2 * w2 - wp
                if 0 <= dj < 5:
                    C[dj, w, wp, w2] = 1.0
    return R, C


_R1, _C1 = _sel1()
_R2, _C2 = _sel2()
_BM1 = np.repeat((np.arange(32) % 16 < 14).astype(np.float32), 8)[None, :]
_BM2 = np.repeat((np.arange(8) < 5).astype(np.float32), 16)[None, :]


def _lenet_batch_kernel(x_ref, wq1_ref, b1_ref, wq2_ref, b2_ref,
                        w3_ref, b3_ref, w4_ref, b4_ref, w5_ref, b5_ref,
                        o_ref):
    B = x_ref.shape[2]

    pieces = [x_ref[ci, u:u + 7] for u in range(2) for ci in range(3)]
    lhs = jnp.concatenate(pieces, axis=2).reshape(7 * B, 768)
    y = jnp.dot(lhs.astype(jnp.bfloat16), wq1_ref[...],
                preferred_element_type=jnp.float32)
    h0 = jnp.maximum(jnp.maximum(y[:, 0:128], y[:, 128:256]),
                     jnp.maximum(y[:, 256:384], y[:, 384:512]))
    h1 = jnp.maximum(jnp.maximum(y[:, 512:640], y[:, 640:768]),
                     jnp.maximum(y[:, 768:896], y[:, 896:1024]))
    a1 = jnp.maximum(jnp.concatenate([h0, h1], axis=1) + b1_ref[...], 0.0)
    a1 = a1.astype(jnp.bfloat16).reshape(7, B, 256)

    lhs2 = jnp.concatenate([a1[t:t + 5] for t in range(3)],
                           axis=2).reshape(5 * B, 768)
    y = jnp.dot(lhs2, wq2_ref[...], preferred_element_type=jnp.float32)
    a2 = jnp.maximum(jnp.maximum(y[:, 0:128], y[:, 128:256]),
                     jnp.maximum(y[:, 256:384], y[:, 384:512]))
    a2 = jnp.maximum(a2 + b2_ref[...], 0.0)
    a2 = a2.reshape(5, B, 128)

    acc = b3_ref[...]
    for h in range(5):
        acc = acc + jnp.dot(a2[h], w3_ref[h],
                            preferred_element_type=jnp.float32)
    f1 = jnp.maximum(acc, 0.0)
    f2 = jnp.maximum(jnp.dot(f1, w4_ref[...],
                             preferred_element_type=jnp.float32) + b4_ref[...],
                     0.0)
    o_ref[...] = jnp.dot(f2, w5_ref[...],
                         preferred_element_type=jnp.float32) + b5_ref[...]


def kernel(x, w1, b1, w2, b2, w3, b3, w4, b4, w5, b5):
    n = x.shape[0]
    xg = x.reshape(n, 3, 8, 128).transpose(1, 2, 0, 3)

    w1t = w1[:, :8].reshape(5, 5, 3, 8)
    wq1 = jnp.einsum("djco,duryh,jwpv->ucrwyhpvo", w1t, _R1, _C1,
                     preferred_element_type=jnp.bfloat16).reshape(768, 1024)
    w2t = jnp.pad(w2[:, :16].reshape(5, 5, 6, 16),
                  ((0, 0), (0, 0), (0, 2), (0, 0)))
    wq2 = jnp.einsum("djco,dtyh,jwpv->tywchpvo", w2t, _R2, _C2,
                     preferred_element_type=jnp.bfloat16).reshape(768, 512)
    w3p = jnp.pad(w3.reshape(5, 80, 128), ((0, 0), (0, 48), (0, 0)))
    b1L = jnp.tile(b1[:, :8], (1, 32)) * _BM1
    b2L = jnp.tile(b2[:, :16], (1, 8)) * _BM2

    grid = n // _B
    c2 = lambda i: (0, 0)
    c3 = lambda i: (0, 0, 0)
    out = pl.pallas_call(
        _lenet_batch_kernel,
        out_shape=jax.ShapeDtypeStruct((n, 128), jnp.float32),
        grid=(grid,),
        in_specs=[
            pl.BlockSpec((3, 8, _B, 128), lambda i: (0, 0, i, 0)),
            pl.BlockSpec((768, 1024), c2),
            pl.BlockSpec((1, 256), c2),
            pl.BlockSpec((768, 512), c2),
            pl.BlockSpec((1, 128), c2),
            pl.BlockSpec((5, 128, 128), c3),
            pl.BlockSpec((1, 128), c2),
            pl.BlockSpec((128, 128), c2),
            pl.BlockSpec((1, 128), c2),
            pl.BlockSpec((128, 128), c2),
            pl.BlockSpec((1, 128), c2),
        ],
        out_specs=pl.BlockSpec((_B, 128), lambda i: (i, 0)),
        compiler_params=pltpu.CompilerParams(
            dimension_semantics=("parallel",),
            vmem_limit_bytes=64 * 1024 * 1024,
        ),
    )(xg, wq1, b1L, wq2, b2L, w3p, b3, w4, b4, w5, b5)

    return out[:, :100]


# speedup vs baseline: 1.6848x; 1.0251x over previous
import numpy as np
import jax
import jax.numpy as jnp
from jax.experimental import pallas as pl
from jax.experimental.pallas import tpu as pltpu

_B = 256


def _sel1():
    R = np.zeros((5, 2, 4, 2, 2), np.float32)
    for u in range(2):
        for r in range(4):
            for r2 in range(2):
                for hp in range(2):
                    di = 4 * u + r - 2 * r2 - hp
                    if 0 <= di < 5:
                        R[di, u, r, r2, hp] = 1.0
    C = np.zeros((5, 32, 2, 16), np.float32)
    for w_in in range(32):
        for wp in range(2):
            for w2 in range(14):
                dj = w_in - 2 * w2 - wp
                if 0 <= dj < 5:
                    C[dj, w_in, wp, w2] = 1.0
    return R, C


def _sel2():
    R = np.zeros((5, 3, 2, 2), np.float32)
    for t in range(3):
        for r2 in range(2):
            for hp in range(2):
                di = 2 * t + r2 - hp
                if 0 <= di < 5:
                    R[di, t, r2, hp] = 1.0
    C = np.zeros((5, 16, 2, 8), np.float32)
    for w in range(14):
        for wp in range(2):
            for w2 in range(5):
                dj = w - 2 * w2 - wp
                if 0 <= dj < 5:
                    C[dj, w, wp, w2] = 1.0
    return R, C


_R1, _C1 = _sel1()
_R2, _C2 = _sel2()
_BM1 = np.repeat((np.arange(32) % 16 < 14).astype(np.float32), 8)[None, :]
_BM2 = np.repeat((np.arange(8) < 5).astype(np.float32), 16)[None, :]


def _lenet_batch_kernel(x_ref, wq1_ref, b1_ref, wq2_ref, b2_ref,
                        w3_ref, b3_ref, w4_ref, b4_ref, w5_ref, b5_ref,
                        o_ref):
    B = x_ref.shape[2]

    pieces = [x_ref[ci, u:u + 7] for u in range(2) for ci in range(3)]
    lhs = jnp.concatenate(pieces, axis=2).reshape(7 * B, 768)
    y = jnp.dot(lhs.astype(jnp.bfloat16), wq1_ref[...],
                preferred_element_type=jnp.float32)
    h0 = jnp.maximum(jnp.maximum(y[:, 0:128], y[:, 128:256]),
                     jnp.maximum(y[:, 256:384], y[:, 384:512]))
    h1 = jnp.maximum(jnp.maximum(y[:, 512:640], y[:, 640:768]),
                     jnp.maximum(y[:, 768:896], y[:, 896:1024]))
    a1 = jnp.maximum(jnp.concatenate([h0, h1], axis=1) + b1_ref[...], 0.0)
    a1 = a1.astype(jnp.bfloat16).reshape(7, B, 256)

    lhs2 = jnp.concatenate([a1[t:t + 5] for t in range(3)],
                           axis=2).reshape(5 * B, 768)
    y = jnp.dot(lhs2, wq2_ref[...], preferred_element_type=jnp.float32)
    a2 = jnp.maximum(jnp.maximum(y[:, 0:128], y[:, 128:256]),
                     jnp.maximum(y[:, 256:384], y[:, 384:512]))
    a2 = jnp.maximum(a2 + b2_ref[...], 0.0)
    a2 = a2.reshape(5, B, 128)

    acc = b3_ref[...]
    for h in range(5):
        acc = acc + jnp.dot(a2[h], w3_ref[h],
                            preferred_element_type=jnp.float32)
    f1 = jnp.maximum(acc, 0.0)
    f2 = jnp.maximum(jnp.dot(f1, w4_ref[...],
                             preferred_element_type=jnp.float32) + b4_ref[...],
                     0.0)
    o_ref[...] = jnp.dot(f2, w5_ref[...],
                         preferred_element_type=jnp.float32) + b5_ref[...]


def kernel(x, w1, b1, w2, b2, w3, b3, w4, b4, w5, b5):
    n = x.shape[0]
    xg = x.reshape(n, 3, 8, 128).transpose(1, 2, 0, 3)

    w1t = w1[:, :8].reshape(5, 5, 3, 8)
    wq1 = jnp.einsum("djco,duryh,jwpv->ucrwyhpvo", w1t, _R1, _C1,
                     preferred_element_type=jnp.bfloat16).reshape(768, 1024)
    w2t = jnp.pad(w2[:, :16].reshape(5, 5, 6, 16),
                  ((0, 0), (0, 0), (0, 2), (0, 0)))
    wq2 = jnp.einsum("djco,dtyh,jwpv->tywchpvo", w2t, _R2, _C2,
                     preferred_element_type=jnp.bfloat16).reshape(768, 512)
    w3p = jnp.pad(w3.reshape(5, 80, 128), ((0, 0), (0, 48), (0, 0)))
    b1L = jnp.tile(b1[:, :8], (1, 32)) * _BM1
    b2L = jnp.tile(b2[:, :16], (1, 8)) * _BM2

    grid = n // _B
    c2 = lambda i: (0, 0)
    c3 = lambda i: (0, 0, 0)
    out = pl.pallas_call(
        _lenet_batch_kernel,
        out_shape=jax.ShapeDtypeStruct((n, 128), jnp.float32),
        grid=(grid,),
        in_specs=[
            pl.BlockSpec((3, 8, _B, 128), lambda i: (0, 0, i, 0)),
            pl.BlockSpec((768, 1024), c2),
            pl.BlockSpec((1, 256), c2),
            pl.BlockSpec((768, 512), c2),
            pl.BlockSpec((1, 128), c2),
            pl.BlockSpec((5, 128, 128), c3),
            pl.BlockSpec((1, 128), c2),
            pl.BlockSpec((128, 128), c2),
            pl.BlockSpec((1, 128), c2),
            pl.BlockSpec((128, 128), c2),
            pl.BlockSpec((1, 128), c2),
        ],
        out_specs=pl.BlockSpec((_B, 128), lambda i: (i, 0)),
        compiler_params=pltpu.CompilerParams(
            dimension_semantics=("parallel",),
            vmem_limit_bytes=64 * 1024 * 1024,
        ),
    )(xg, wq1, b1L, wq2, b2L, w3p, b3, w4, b4, w5, b5)

    return out[:, :100]


# speedup vs baseline: 1.7042x; 1.0115x over previous
import numpy as np
import jax
import jax.numpy as jnp
from jax.experimental import pallas as pl
from jax.experimental.pallas import tpu as pltpu

_B = 512


def _sel1():
    R = np.zeros((5, 2, 4, 2, 2), np.float32)
    for u in range(2):
        for r in range(4):
            for r2 in range(2):
                for hp in range(2):
                    di = 4 * u + r - 2 * r2 - hp
                    if 0 <= di < 5:
                        R[di, u, r, r2, hp] = 1.0
    C = np.zeros((5, 32, 2, 16), np.float32)
    for w_in in range(32):
        for wp in range(2):
            for w2 in range(14):
                dj = w_in - 2 * w2 - wp
                if 0 <= dj < 5:
                    C[dj, w_in, wp, w2] = 1.0
    return R, C


def _sel2():
    R = np.zeros((5, 3, 2, 2), np.float32)
    for t in range(3):
        for r2 in range(2):
            for hp in range(2):
                di = 2 * t + r2 - hp
                if 0 <= di < 5:
                    R[di, t, r2, hp] = 1.0
    C = np.zeros((5, 16, 2, 8), np.float32)
    for w in range(14):
        for wp in range(2):
            for w2 in range(5):
                dj = w - 2 * w2 - wp
                if 0 <= dj < 5:
                    C[dj, w, wp, w2] = 1.0
    return R, C


_R1, _C1 = _sel1()
_R2, _C2 = _sel2()
_BM1 = np.repeat((np.arange(32) % 16 < 14).astype(np.float32), 8)[None, :]
_BM2 = np.repeat((np.arange(8) < 5).astype(np.float32), 16)[None, :]


def _lenet_batch_kernel(x_ref, wq1_ref, b1_ref, wq2_ref, b2_ref,
                        w3_ref, b3_ref, w4_ref, b4_ref, w5_ref, b5_ref,
                        o_ref):
    B = x_ref.shape[2]

    pieces = [x_ref[ci, u:u + 7] for u in range(2) for ci in range(3)]
    lhs = jnp.concatenate(pieces, axis=2).reshape(7 * B, 768)
    y = jnp.dot(lhs.astype(jnp.bfloat16), wq1_ref[...],
                preferred_element_type=jnp.float32)
    h0 = jnp.maximum(jnp.maximum(y[:, 0:128], y[:, 128:256]),
                     jnp.maximum(y[:, 256:384], y[:, 384:512]))
    h1 = jnp.maximum(jnp.maximum(y[:, 512:640], y[:, 640:768]),
                     jnp.maximum(y[:, 768:896], y[:, 896:1024]))
    a1 = jnp.maximum(jnp.concatenate([h0, h1], axis=1) + b1_ref[...], 0.0)
    a1 = a1.astype(jnp.bfloat16).reshape(7, B, 256)

    lhs2 = jnp.concatenate([a1[t:t + 5] for t in range(3)],
                           axis=2).reshape(5 * B, 768)
    y = jnp.dot(lhs2, wq2_ref[...], preferred_element_type=jnp.float32)
    a2 = jnp.maximum(jnp.maximum(y[:, 0:128], y[:, 128:256]),
                     jnp.maximum(y[:, 256:384], y[:, 384:512]))
    a2 = jnp.maximum(a2 + b2_ref[...], 0.0)
    a2 = a2.reshape(5, B, 128)

    acc = b3_ref[...]
    for h in range(5):
        acc = acc + jnp.dot(a2[h], w3_ref[h],
                            preferred_element_type=jnp.float32)
    f1 = jnp.maximum(acc, 0.0)
    f2 = jnp.maximum(jnp.dot(f1, w4_ref[...],
                             preferred_element_type=jnp.float32) + b4_ref[...],
                     0.0)
    o_ref[...] = jnp.dot(f2, w5_ref[...],
                         preferred_element_type=jnp.float32) + b5_ref[...]


def kernel(x, w1, b1, w2, b2, w3, b3, w4, b4, w5, b5):
    n = x.shape[0]
    xg = x.reshape(n, 3, 8, 128).transpose(1, 2, 0, 3)

    w1t = w1[:, :8].reshape(5, 5, 3, 8)
    wq1 = jnp.einsum("djco,duryh,jwpv->ucrwyhpvo", w1t, _R1, _C1,
                     preferred_element_type=jnp.bfloat16).reshape(768, 1024)
    w2t = jnp.pad(w2[:, :16].reshape(5, 5, 6, 16),
                  ((0, 0), (0, 0), (0, 2), (0, 0)))
    wq2 = jnp.einsum("djco,dtyh,jwpv->tywchpvo", w2t, _R2, _C2,
                     preferred_element_type=jnp.bfloat16).reshape(768, 512)
    w3p = jnp.pad(w3.reshape(5, 80, 128), ((0, 0), (0, 48), (0, 0)))
    b1L = jnp.tile(b1[:, :8], (1, 32)) * _BM1
    b2L = jnp.tile(b2[:, :16], (1, 8)) * _BM2

    grid = n // _B
    c2 = lambda i: (0, 0)
    c3 = lambda i: (0, 0, 0)
    out = pl.pallas_call(
        _lenet_batch_kernel,
        out_shape=jax.ShapeDtypeStruct((n, 128), jnp.float32),
        grid=(grid,),
        in_specs=[
            pl.BlockSpec((3, 8, _B, 128), lambda i: (0, 0, i, 0)),
            pl.BlockSpec((768, 1024), c2),
            pl.BlockSpec((1, 256), c2),
            pl.BlockSpec((768, 512), c2),
            pl.BlockSpec((1, 128), c2),
            pl.BlockSpec((5, 128, 128), c3),
            pl.BlockSpec((1, 128), c2),
            pl.BlockSpec((128, 128), c2),
            pl.BlockSpec((1, 128), c2),
            pl.BlockSpec((128, 128), c2),
            pl.BlockSpec((1, 128), c2),
        ],
        out_specs=pl.BlockSpec((_B, 128), lambda i: (i, 0)),
        compiler_params=pltpu.CompilerParams(
            dimension_semantics=("parallel",),
            vmem_limit_bytes=64 * 1024 * 1024,
        ),
    )(xg, wq1, b1L, wq2, b2L, w3p, b3, w4, b4, w5, b5)

    return out[:, :100]
